# baseline (speedup 1.0000x reference)
import functools

import jax
import jax.numpy as jnp
from jax.experimental import pallas as pl
from jax.experimental.pallas import tpu as pltpu


def _moe_kernel(x_ref, c_ref, W1_ref, b1_ref, W2_ref, b2_ref,
                Ws_ref, bs_ref, WmR_ref, WvR_ref, bmv_ref,
                C1T_ref, C2T_ref, Rep_ref, Csum_ref,
                refined_ref, delta_ref, loss_ref,
                wbig_s, wbd_s, bsf_s, *, n_tokens, n_experts, hdim):
    E, H = n_experts, hdim
    EH = E * H

    @pl.when(pl.program_id(0) == 0)
    def _prep():
        for e in range(E):
            wbig_s[:, e * H:(e + 1) * H] = Ws_ref[e].astype(jnp.bfloat16)
        sub_e = jax.lax.broadcasted_iota(jnp.int32, (EH, 4 * E), 0) // H
        lane_e = jax.lax.broadcasted_iota(jnp.int32, (EH, 4 * E), 1) // 4
        wmv32 = (jnp.dot(WmR_ref[...], C1T_ref[...],
                         preferred_element_type=jnp.float32)
                 + jnp.dot(WvR_ref[...], C2T_ref[...],
                           preferred_element_type=jnp.float32))
        wbd_s[...] = jnp.where(sub_e == lane_e, wmv32, 0.0).astype(jnp.bfloat16)
        bsf_s[...] = jnp.concatenate(
            [bs_ref[e:e + 1, :] for e in range(E)], axis=1)

    x = x_ref[...]

    big = jnp.dot(x.astype(jnp.bfloat16), wbig_s[...],
                  preferred_element_type=jnp.float32)
    feats = jnp.maximum(big + bsf_s[...], 0.0)

    h1 = jnp.maximum(jnp.dot(x, W1_ref[...],
                             preferred_element_type=jnp.float32) + b1_ref[...], 0.0)
    zT = jax.lax.dot_general(W2_ref[...], h1, (((0,), (1,)), ((), ())),
                             preferred_element_type=jnp.float32) + b2_ref[...]
    zT = zT - jnp.max(zT, axis=0, keepdims=True)
    ez = jnp.exp(zT)
    S = jnp.sum(ez, axis=0, keepdims=True)

    ent = jnp.log(S) - jnp.sum(ez * zT, axis=0, keepdims=True) / S
    part = (jnp.sum(ent) / n_tokens).reshape(1, 1)

    @pl.when(pl.program_id(0) == 0)
    def _():
        loss_ref[...] = jnp.zeros((1, 1), jnp.float32)

    loss_ref[...] += part

    v1 = jnp.max(ez, axis=0, keepdims=True)
    m1 = ez >= v1
    c1 = jnp.sum(m1.astype(jnp.float32), axis=0, keepdims=True)
    vr = jnp.max(jnp.where(m1, -1.0, ez), axis=0, keepdims=True)
    v2 = jnp.where(c1 > 1.0, v1, vr)
    wT = jnp.where(ez >= v2, ez, 0.0) / (v1 + v2)

    outs = jnp.dot(feats.astype(jnp.bfloat16), wbd_s[...],
                   preferred_element_type=jnp.float32) + bmv_ref[...]
    T = x.shape[0]
    EC = outs.shape[1]
    c_iota = jax.lax.broadcasted_iota(jnp.int32, (T, EC), 1)
    is_mean = (c_iota % 4) < 2
    sp = jnp.maximum(outs, 0.0) + jnp.log1p(jnp.exp(-jnp.abs(outs)))
    acts = jnp.where(is_mean, jnp.tanh(outs), sp)

    w32 = jax.lax.dot_general(wT, Rep_ref[...], (((0,), (0,)), ((), ())),
                              preferred_element_type=jnp.float32)
    delta = jnp.dot(acts * w32, Csum_ref[...],
                    preferred_element_type=jnp.float32)
    delta_ref[...] = delta
    refined_ref[...] = jnp.clip(c_ref[...] + delta[:, :2] * 0.002, 0.0, 1.0)


def kernel(x, coarse_coord, W1, b1, W2, b2, Ws, bs, Wm, bm, Wv, bv):
    B, S, D = x.shape
    GH = W1.shape[1]
    E = W2.shape[1]
    H = Ws.shape[2]
    N = B * S
    T = 1024
    grid = N // T
    EH = E * H

    x2 = x.reshape(N, D)
    c2 = coarse_coord.reshape(N, 2)
    WmR = Wm.reshape(EH, 2)
    WvR = Wv.reshape(EH, 2)
    bmvf = jnp.concatenate([bm, bv], axis=-1).reshape(1, E * 4)
    b1r = b1.reshape(1, GH)
    b2c = b2.reshape(E, 1)
    f32 = jnp.float32
    lane = jnp.arange(4 * E)
    C1T = jnp.stack([(lane % 4 == 0).astype(f32), (lane % 4 == 1).astype(f32)])
    C2T = jnp.stack([(lane % 4 == 2).astype(f32), (lane % 4 == 3).astype(f32)])
    Rep = jnp.kron(jnp.eye(E, dtype=f32), jnp.ones((1, 4), f32))
    Csum = jnp.tile(jnp.eye(4, dtype=f32), (E, 1))

    body = functools.partial(_moe_kernel, n_tokens=float(N),
                             n_experts=E, hdim=H)
    full = lambda i: (0, 0)
    full3 = lambda i: (0, 0, 0)
    refined, delta, loss = pl.pallas_call(
        body,
        grid=(grid,),
        in_specs=[
            pl.BlockSpec((T, D), lambda i: (i, 0)),
            pl.BlockSpec((T, 2), lambda i: (i, 0)),
            pl.BlockSpec((D, GH), full),
            pl.BlockSpec((1, GH), full),
            pl.BlockSpec((GH, E), full),
            pl.BlockSpec((E, 1), full),
            pl.BlockSpec((E, D, H), full3),
            pl.BlockSpec((E, H), full),
            pl.BlockSpec((EH, 2), full),
            pl.BlockSpec((EH, 2), full),
            pl.BlockSpec((1, E * 4), full),
            pl.BlockSpec((2, E * 4), full),
            pl.BlockSpec((2, E * 4), full),
            pl.BlockSpec((E, E * 4), full),
            pl.BlockSpec((E * 4, 4), full),
        ],
        out_specs=[
            pl.BlockSpec((T, 2), lambda i: (i, 0)),
            pl.BlockSpec((T, 4), lambda i: (i, 0)),
            pl.BlockSpec((1, 1), full),
        ],
        out_shape=[
            jax.ShapeDtypeStruct((N, 2), jnp.float32),
            jax.ShapeDtypeStruct((N, 4), jnp.float32),
            jax.ShapeDtypeStruct((1, 1), jnp.float32),
        ],
        scratch_shapes=[
            pltpu.VMEM((D, EH), jnp.bfloat16),
            pltpu.VMEM((EH, 4 * E), jnp.bfloat16),
            pltpu.VMEM((1, EH), jnp.float32),
        ],
    )(x2, c2, W1, b1r, W2, b2c, Ws, bs, WmR, WvR, bmvf, C1T, C2T, Rep, Csum)

    return (refined.reshape(B, S, 2), loss[0, 0], delta.reshape(B, S, 4))


# speedup vs baseline: 1.0604x; 1.0604x over previous
import functools

import jax
import jax.numpy as jnp
from jax.experimental import pallas as pl


def _moe_kernel(x_ref, c_ref, W1_ref, b1_ref, W2_ref, b2_ref,
                Wbig_ref, bsf_ref, Wbd_ref, bmv_ref, Rep_ref, Csum_ref,
                refined_ref, delta_ref, loss_ref, *, n_tokens, eh):
    x = x_ref[...]

    big = jnp.dot(x.astype(jnp.bfloat16), Wbig_ref[...],
                  preferred_element_type=jnp.float32)
    feats = jnp.maximum(big[:, :eh] + bsf_ref[...], 0.0)

    h1 = jnp.maximum(jnp.dot(x, W1_ref[...],
                             preferred_element_type=jnp.float32) + b1_ref[...], 0.0)
    zT = jax.lax.dot_general(W2_ref[...], h1, (((0,), (1,)), ((), ())),
                             preferred_element_type=jnp.float32) + b2_ref[...]
    zT = zT - jnp.max(zT, axis=0, keepdims=True)
    ez = jnp.exp(zT)
    S = jnp.sum(ez, axis=0, keepdims=True)

    ent = jnp.log(S) - jnp.sum(ez * zT, axis=0, keepdims=True) / S
    part = (jnp.sum(ent) / n_tokens).reshape(1, 1)

    @pl.when(pl.program_id(0) == 0)
    def _():
        loss_ref[...] = jnp.zeros((1, 1), jnp.float32)

    loss_ref[...] += part

    v1 = jnp.max(ez, axis=0, keepdims=True)
    m1 = ez >= v1
    c1 = jnp.sum(m1.astype(jnp.float32), axis=0, keepdims=True)
    vr = jnp.max(jnp.where(m1, -1.0, ez), axis=0, keepdims=True)
    v2 = jnp.where(c1 > 1.0, v1, vr)
    wT = jnp.where(ez >= v2, ez, 0.0) / (v1 + v2)

    outs = jnp.dot(feats.astype(jnp.bfloat16), Wbd_ref[...],
                   preferred_element_type=jnp.float32) + bmv_ref[...]
    T = x.shape[0]
    EC = outs.shape[1]
    c_iota = jax.lax.broadcasted_iota(jnp.int32, (T, EC), 1)
    is_mean = (c_iota % 4) < 2
    sp = jnp.maximum(outs, 0.0) + jnp.log1p(jnp.exp(-jnp.abs(outs)))
    acts = jnp.where(is_mean, jnp.tanh(outs), sp)

    w32 = jax.lax.dot_general(wT, Rep_ref[...], (((0,), (0,)), ((), ())),
                              preferred_element_type=jnp.float32)
    delta = jnp.dot(acts * w32, Csum_ref[...],
                    preferred_element_type=jnp.float32)
    delta_ref[...] = delta
    refined_ref[...] = jnp.clip(c_ref[...] + delta[:, :2] * 0.002, 0.0, 1.0)


def kernel(x, coarse_coord, W1, b1, W2, b2, Ws, bs, Wm, bm, Wv, bv):
    B, S, D = x.shape
    GH = W1.shape[1]
    E = W2.shape[1]
    H = Ws.shape[2]
    N = B * S
    T = 2048
    grid = N // T
    EH = E * H

    x2 = x.reshape(N, D)
    c2 = coarse_coord.reshape(N, 2)
    Wall = jnp.transpose(Ws, (1, 0, 2)).reshape(D, EH)
    Wbig = Wall.astype(jnp.bfloat16)
    bsf = bs.reshape(1, EH)
    Wmv = jnp.concatenate([Wm, Wv], axis=-1)
    eye = jnp.eye(E, dtype=x.dtype)
    Wbd = (eye[:, None, :, None] * Wmv[:, :, None, :]).reshape(EH, E * 4).astype(jnp.bfloat16)
    bmvf = jnp.concatenate([bm, bv], axis=-1).reshape(1, E * 4)
    Rep = jnp.kron(jnp.eye(E, dtype=x.dtype), jnp.ones((1, 4), x.dtype))
    Csum = jnp.tile(jnp.eye(4, dtype=x.dtype), (E, 1))
    b1r = b1.reshape(1, GH)
    b2c = b2.reshape(E, 1)

    body = functools.partial(_moe_kernel, n_tokens=float(N), eh=EH)
    full = lambda i: (0, 0)
    refined, delta, loss = pl.pallas_call(
        body,
        grid=(grid,),
        in_specs=[
            pl.BlockSpec((T, D), lambda i: (i, 0)),
            pl.BlockSpec((T, 2), lambda i: (i, 0)),
            pl.BlockSpec((D, GH), full),
            pl.BlockSpec((1, GH), full),
            pl.BlockSpec((GH, E), full),
            pl.BlockSpec((E, 1), full),
            pl.BlockSpec((D, EH), full),
            pl.BlockSpec((1, EH), full),
            pl.BlockSpec((EH, E * 4), full),
            pl.BlockSpec((1, E * 4), full),
            pl.BlockSpec((E, E * 4), full),
            pl.BlockSpec((E * 4, 4), full),
        ],
        out_specs=[
            pl.BlockSpec((T, 2), lambda i: (i, 0)),
            pl.BlockSpec((T, 4), lambda i: (i, 0)),
            pl.BlockSpec((1, 1), full),
        ],
        out_shape=[
            jax.ShapeDtypeStruct((N, 2), jnp.float32),
            jax.ShapeDtypeStruct((N, 4), jnp.float32),
            jax.ShapeDtypeStruct((1, 1), jnp.float32),
        ],
    )(x2, c2, W1, b1r, W2, b2c, Wbig, bsf, Wbd, bmvf, Rep, Csum)

    return (refined.reshape(B, S, 2), loss[0, 0], delta.reshape(B, S, 4))
